# Initial kernel scaffold; baseline (speedup 1.0000x reference)
import jax
import jax.numpy as jnp
from jax.experimental import pallas as pl


def kernel(xyz, features, W1, g1, b1, W2, g2, b2, W3, g3, b3):
    raise NotImplementedError("write your pallas kernel here")


# speedup vs baseline: 17.7709x; 17.7709x over previous
import functools

import jax
import jax.numpy as jnp
from jax import lax
from jax.experimental import pallas as pl
from jax.experimental.pallas import tpu as pltpu
from jax.experimental.pallas import tpu_sc as plsc

B, N, C = 4, 8192, 64
NPOINT, RADIUS, NSAMPLE = 1024, 0.4, 32
EPS = 1e-5
NROW, NCOL = 64, 128


def _fps_body(nb, nrow, ncol, npoint, xyz_ref, nx_ref, ny_ref, nz_ref):
    X = xyz_ref[:, 0]
    Y = xyz_ref[:, 1]
    Z = xyz_ref[:, 2]
    flat = (lax.broadcasted_iota(jnp.int32, (nb, nrow, ncol), 1) * ncol
            + lax.broadcasted_iota(jnp.int32, (nb, nrow, ncol), 2))
    pos = lax.broadcasted_iota(jnp.int32, (nb, npoint), 1)
    big = jnp.int32(1 << 30)

    def body(i, st):
        dists, far, nxa, nya, nza = st
        onehot = flat == far[:, :, None]
        cx = jnp.sum(jnp.sum(jnp.where(onehot, X, 0.0), axis=2), axis=1)
        cy = jnp.sum(jnp.sum(jnp.where(onehot, Y, 0.0), axis=2), axis=1)
        cz = jnp.sum(jnp.sum(jnp.where(onehot, Z, 0.0), axis=2), axis=1)
        sel = pos == i
        nxa = jnp.where(sel, cx[:, None], nxa)
        nya = jnp.where(sel, cy[:, None], nya)
        nza = jnp.where(sel, cz[:, None], nza)
        dx = X - cx[:, None, None]
        dy = Y - cy[:, None, None]
        dz = Z - cz[:, None, None]
        d = dx * dx + dy * dy
        d = d + dz * dz
        dists = jnp.minimum(dists, d)
        m = jnp.max(jnp.max(dists, axis=2), axis=1)
        cand = jnp.where(dists == m[:, None, None], flat, big)
        far_new = jnp.min(jnp.min(cand, axis=2), axis=1)
        return dists, far_new[:, None], nxa, nya, nza

    dists0 = jnp.full((nb, nrow, ncol), 1e10, dtype=jnp.float32)
    far0 = jnp.zeros((nb, 1), dtype=jnp.int32)
    z = jnp.zeros((nb, npoint), dtype=jnp.float32)
    _, _, nxa, nya, nza = lax.fori_loop(0, npoint, body, (dists0, far0, z, z, z))
    nx_ref[...] = nxa
    ny_ref[...] = nya
    nz_ref[...] = nza


def _fps_call(xyz_p):
    nb = xyz_p.shape[0]
    out = jax.ShapeDtypeStruct((nb, NPOINT), jnp.float32)
    return pl.pallas_call(
        functools.partial(_fps_body, nb, NROW, NCOL, NPOINT),
        out_shape=[out, out, out],
    )(xyz_p)


def _pts_body(p_ref, w_ref, a_ref):
    a_ref[0] = jnp.dot(p_ref[0], w_ref[...], preferred_element_type=jnp.float32)


def _pts_call(pts, W1s):
    nb, n, ci = pts.shape
    co = W1s.shape[1]
    return pl.pallas_call(
        _pts_body,
        grid=(nb,),
        in_specs=[
            pl.BlockSpec((1, n, ci), lambda i: (i, 0, 0)),
            pl.BlockSpec((ci, co), lambda i: (0, 0)),
        ],
        out_specs=pl.BlockSpec((1, n, co), lambda i: (i, 0, 0)),
        out_shape=jax.ShapeDtypeStruct((nb, n, co), jnp.float32),
    )(pts, W1s)


def _sc_body(xs_hbm, ys_hbm, zs_hbm, cx_hbm, cy_hbm, cz_hbm, table_hbm, out_hbm,
             xs_v, ys_v, zs_v, cent_v, gbuf, idxbuf, rows_v, sem):
    nw = 32
    s_per_w = (B * NPOINT) // nw
    wid = lax.axis_index("s") * 2 + lax.axis_index("c")
    b = wid // (nw // B)
    cbase = wid * s_per_w
    r2 = jnp.float32(RADIUS * RADIUS)
    iota = lax.iota(jnp.int32, 16)

    pltpu.sync_copy(xs_hbm.at[pl.ds(b * N, N)], xs_v)
    pltpu.sync_copy(ys_hbm.at[pl.ds(b * N, N)], ys_v)
    pltpu.sync_copy(zs_hbm.at[pl.ds(b * N, N)], zs_v)
    for c, ref in enumerate((cx_hbm, cy_hbm, cz_hbm)):
        pltpu.sync_copy(ref.at[pl.ds(cbase, s_per_w)],
                        cent_v.at[pl.ds(c * s_per_w, s_per_w)])

    def per_centroid(s, carry):
        base16 = (s // 16) * 16
        lanem = iota == (s - base16)
        neg = jnp.float32(-3e38)

        def bcast(off):
            chunk = cent_v[pl.ds(off + base16, 16)]
            return jnp.full((16,), jnp.max(jnp.where(lanem, chunk, neg)))

        cxv = bcast(0)
        cyv = bcast(s_per_w)
        czv = bcast(2 * s_per_w)

        def scan_cond(jc):
            j, cnt = jc
            return jnp.logical_and(cnt < NSAMPLE, j < N)

        def scan_body(jc):
            j, cnt = jc
            for u in range(4):
                jj = j + 16 * u
                px = xs_v[pl.ds(jj, 16)]
                py = ys_v[pl.ds(jj, 16)]
                pz = zs_v[pl.ds(jj, 16)]
                dx = px - cxv
                dy = py - cyv
                dz = pz - czv
                d2 = dx * dx + dy * dy
                d2 = d2 + dz * dz
                mask = d2 < r2
                incl = jnp.cumsum(jnp.where(mask, 1, 0).astype(jnp.int32))
                posn = cnt + incl - 1
                plsc.store_scatter(gbuf, [posn], jj + iota, mask=mask)
                cnt = cnt + jnp.max(incl)
            return j + 64, cnt

        _, cnt = lax.while_loop(scan_cond, scan_body, (jnp.int32(0), jnp.int32(0)))

        v0 = gbuf[pl.ds(0, 16)]
        v1 = gbuf[pl.ds(16, 16)]
        first = jnp.max(jnp.where(iota == 0, v0, -1))
        fb = jnp.full((16,), first, jnp.int32)
        boff = b * N
        out0 = jnp.where(iota < cnt, v0, fb) + boff
        out1 = jnp.where(iota + 16 < cnt, v1, fb) + boff
        idxbuf[pl.ds(0, 16)] = out0
        idxbuf[pl.ds(16, 16)] = out1

        pltpu.async_copy(table_hbm.at[idxbuf], rows_v, sem).wait()
        rowbase = (cbase + s) * NSAMPLE
        pltpu.sync_copy(rows_v, out_hbm.at[pl.ds(rowbase, NSAMPLE)])
        return carry

    lax.fori_loop(0, s_per_w, per_centroid, jnp.int32(0))


def _sc_call(coords, cents, table):
    mesh = plsc.VectorSubcoreMesh(core_axis_name="c", subcore_axis_name="s")
    fn = pl.kernel(
        _sc_body,
        mesh=mesh,
        compiler_params=pltpu.CompilerParams(needs_layout_passes=False),
        out_type=jax.ShapeDtypeStruct((B * NPOINT * NSAMPLE, 128), jnp.float32),
        scratch_types=[
            pltpu.VMEM((N,), jnp.float32),
            pltpu.VMEM((N,), jnp.float32),
            pltpu.VMEM((N,), jnp.float32),
            pltpu.VMEM((3 * 128,), jnp.float32),
            pltpu.VMEM((96,), jnp.int32),
            pltpu.VMEM((NSAMPLE,), jnp.int32),
            pltpu.VMEM((NSAMPLE, 128), jnp.float32),
            pltpu.SemaphoreType.DMA,
        ],
    )
    xs, ys, zs = (coords[:, i, :].reshape(B * N) for i in range(3))
    return fn(xs, ys, zs, cents[0], cents[1], cents[2], table)


def _mlp_body(sblk, g_ref, c_ref, w1c_ref, b1_ref, w2_ref, b2_ref,
              w3_ref, b3_ref, o_ref):
    q = lax.dot_general(c_ref[...], w1c_ref[...], (((0,), (0,)), ((), ())),
                        preferred_element_type=jnp.float32)
    qb = q - b1_ref[...]
    g3 = g_ref[...].reshape(sblk, NSAMPLE, 128)
    x1 = jnp.maximum(g3 - qb[:, None, :], 0.0).reshape(sblk * NSAMPLE, 128)
    x2 = jnp.maximum(
        jnp.dot(x1, w2_ref[...], preferred_element_type=jnp.float32)
        + b2_ref[...], 0.0)
    x3 = jnp.maximum(
        jnp.dot(x2, w3_ref[...], preferred_element_type=jnp.float32)
        + b3_ref[...], 0.0)
    o_ref[...] = jnp.max(x3.reshape(sblk, NSAMPLE, 128), axis=1)


def _mlp_call(G, cents, W1cs, b1r, W2s, b2r, W3s, b3r):
    sblk = 256
    ns = NSAMPLE
    stot = B * NPOINT
    grid = (stot // sblk,)
    return pl.pallas_call(
        functools.partial(_mlp_body, sblk),
        grid=grid,
        in_specs=[
            pl.BlockSpec((sblk * ns, 128), lambda i: (i, 0)),
            pl.BlockSpec((3, sblk), lambda i: (0, i)),
            pl.BlockSpec((3, 128), lambda i: (0, 0)),
            pl.BlockSpec((1, 128), lambda i: (0, 0)),
            pl.BlockSpec((128, 64), lambda i: (0, 0)),
            pl.BlockSpec((1, 64), lambda i: (0, 0)),
            pl.BlockSpec((64, 128), lambda i: (0, 0)),
            pl.BlockSpec((1, 128), lambda i: (0, 0)),
        ],
        out_specs=pl.BlockSpec((sblk, 128), lambda i: (i, 0)),
        out_shape=jax.ShapeDtypeStruct((stot, 128), jnp.float32),
    )(G, cents, W1cs, b1r, W2s, b2r, W3s, b3r)


def kernel(xyz, features, W1, g1, b1, W2, g2, b2, W3, g3, b3):
    k = (1.0 / jnp.sqrt(jnp.float32(1.0 + EPS))).astype(jnp.float32)
    coords = xyz.transpose(0, 2, 1)
    xyz_p = coords.reshape(B, 3, NROW, NCOL)

    nx, ny, nz = _fps_call(xyz_p)
    new_xyz = jnp.stack([nx, ny, nz], axis=-1)
    cents = jnp.concatenate(
        [nx.reshape(1, B * NPOINT), ny.reshape(1, B * NPOINT),
         nz.reshape(1, B * NPOINT)], axis=0)

    feats_t = features.transpose(0, 2, 1)
    pts = jnp.concatenate([xyz, feats_t], axis=-1)
    scale1 = (k * g1)[:, None]
    W1s = jnp.pad((W1 * scale1).T, ((0, 0), (0, 64)))
    A = _pts_call(pts, W1s).reshape(B * N, 128)

    G = _sc_call(coords, cents, A)

    W1cs = jnp.pad((W1[:, :3] * scale1).T, ((0, 0), (0, 64)))
    W2s = jnp.pad((W2 * (k * g2)[:, None]).T, ((0, 64), (0, 0)))
    W3s = (W3 * (k * g3)[:, None]).T
    out = _mlp_call(G, cents, W1cs, jnp.pad(b1, (0, 64)).reshape(1, 128), W2s,
                    b2.reshape(1, 64), W3s, b3.reshape(1, 128))
    new_features = out.reshape(B, NPOINT, 128).transpose(0, 2, 1)
    return new_xyz, new_features


# speedup vs baseline: 25.2460x; 1.4206x over previous
import functools

import jax
import jax.numpy as jnp
from jax import lax
from jax.experimental import pallas as pl
from jax.experimental.pallas import tpu as pltpu
from jax.experimental.pallas import tpu_sc as plsc

B, N, C = 4, 8192, 64
NPOINT, RADIUS, NSAMPLE = 1024, 0.4, 32
EPS = 1e-5
NROW, NCOL = 64, 128


def _fps_body(nb, nrow, ncol, npoint, xyz_ref, nx_ref, ny_ref, nz_ref,
              dists_ref):
    big = jnp.int32(1 << 30)
    rows1 = lax.broadcasted_iota(jnp.int32, (nrow, ncol), 0)
    col8 = lax.broadcasted_iota(jnp.int32, (8, ncol), 1)
    lane = lax.broadcasted_iota(jnp.int32, (1, ncol), 1)

    def comb(ta, tb):
        da, ra, xa, ya, za = ta
        db, rb, xb, yb, zb = tb
        take = (da > db) | ((da == db) & (ra < rb))
        f = lambda u, v: jnp.where(take, u, v)
        return f(da, db), f(ra, rb), f(xa, xb), f(ya, yb), f(za, zb)

    def tree_argmax(D, X, Y, Z):
        t = (D, rows1, X, Y, Z)
        r = nrow
        while r > 8:
            h = r // 2
            t = comb(tuple(a[:h] for a in t), tuple(a[h:] for a in t))
            r = h
        Ds, Rs, Xs, Ys, Zs = t
        def red2(op, a):
            return op(op(a, axis=0, keepdims=True), axis=1, keepdims=True)

        fl = Rs * ncol + col8
        m = red2(jnp.max, Ds)
        cand = jnp.where(Ds == m, fl, big)
        wi = red2(jnp.min, cand)
        sel = fl == wi
        wx = red2(jnp.sum, jnp.where(sel, Xs, 0.0))
        wy = red2(jnp.sum, jnp.where(sel, Ys, 0.0))
        wz = red2(jnp.sum, jnp.where(sel, Zs, 0.0))
        return wx, wy, wz

    def body(i, st):
        ws, chunks = st
        sel = lane == (i & (ncol - 1))
        k = i >> 7
        new_ws = []
        new_chunks = []
        for b in range(nb):
            wx, wy, wz = ws[b]
            cx, cy, cz = chunks[b]
            cx = jnp.where(sel, wx, cx)
            cy = jnp.where(sel, wy, cy)
            cz = jnp.where(sel, wz, cz)
            nx_ref[b, pl.ds(k, 1)] = cx.reshape(1, 1, ncol)
            ny_ref[b, pl.ds(k, 1)] = cy.reshape(1, 1, ncol)
            nz_ref[b, pl.ds(k, 1)] = cz.reshape(1, 1, ncol)
            new_chunks.append((cx, cy, cz))
            X = xyz_ref[b, 0]
            Y = xyz_ref[b, 1]
            Z = xyz_ref[b, 2]
            dx = X - wx
            dy = Y - wy
            dz = Z - wz
            d = dx * dx + dy * dy
            d = d + dz * dz
            dists = jnp.minimum(dists_ref[b], d)
            dists_ref[b] = dists
            new_ws.append(tree_argmax(dists, X, Y, Z))
        return tuple(new_ws), tuple(new_chunks)

    ws0 = []
    for b in range(nb):
        dists_ref[b] = jnp.full((nrow, ncol), 1e10, dtype=jnp.float32)
        p0 = (rows1 * ncol + lax.broadcasted_iota(jnp.int32, (nrow, ncol), 1)) == 0

        def red0(a):
            return jnp.sum(jnp.sum(a, axis=0, keepdims=True), axis=1,
                           keepdims=True)

        wx0 = red0(jnp.where(p0, xyz_ref[b, 0], 0.0))
        wy0 = red0(jnp.where(p0, xyz_ref[b, 1], 0.0))
        wz0 = red0(jnp.where(p0, xyz_ref[b, 2], 0.0))
        ws0.append((wx0, wy0, wz0))
    zc = jnp.zeros((1, ncol), dtype=jnp.float32)
    st = (tuple(ws0), tuple((zc, zc, zc) for _ in range(nb)))
    lax.fori_loop(0, npoint, body, st)


def _fps_call(xyz_p):
    nb = xyz_p.shape[0]
    out = jax.ShapeDtypeStruct((nb, NPOINT // NCOL, 1, NCOL), jnp.float32)
    nx, ny, nz = pl.pallas_call(
        functools.partial(_fps_body, nb, NROW, NCOL, NPOINT),
        out_shape=[out, out, out],
        scratch_shapes=[pltpu.VMEM((nb, NROW, NCOL), jnp.float32)],
    )(xyz_p)
    return (nx.reshape(nb, NPOINT), ny.reshape(nb, NPOINT),
            nz.reshape(nb, NPOINT))


def _pts_body(p_ref, w_ref, a_ref):
    a_ref[0] = jnp.dot(p_ref[0], w_ref[...], preferred_element_type=jnp.float32)


def _pts_call(pts, W1s):
    nb, n, ci = pts.shape
    co = W1s.shape[1]
    return pl.pallas_call(
        _pts_body,
        grid=(nb,),
        in_specs=[
            pl.BlockSpec((1, n, ci), lambda i: (i, 0, 0)),
            pl.BlockSpec((ci, co), lambda i: (0, 0)),
        ],
        out_specs=pl.BlockSpec((1, n, co), lambda i: (i, 0, 0)),
        out_shape=jax.ShapeDtypeStruct((nb, n, co), jnp.float32),
    )(pts, W1s)


def _sc_body(xs_hbm, ys_hbm, zs_hbm, cx_hbm, cy_hbm, cz_hbm, table_hbm, out_hbm,
             xs_v, ys_v, zs_v, cent_v, gbuf, idxbuf, rows_v, sem):
    nw = 32
    s_per_w = (B * NPOINT) // nw
    wid = lax.axis_index("s") * 2 + lax.axis_index("c")
    b = wid // (nw // B)
    cbase = wid * s_per_w
    r2 = jnp.float32(RADIUS * RADIUS)
    iota = lax.iota(jnp.int32, 16)

    pltpu.sync_copy(xs_hbm.at[pl.ds(b * N, N)], xs_v)
    pltpu.sync_copy(ys_hbm.at[pl.ds(b * N, N)], ys_v)
    pltpu.sync_copy(zs_hbm.at[pl.ds(b * N, N)], zs_v)
    for c, ref in enumerate((cx_hbm, cy_hbm, cz_hbm)):
        pltpu.sync_copy(ref.at[pl.ds(cbase, s_per_w)],
                        cent_v.at[pl.ds(c * s_per_w, s_per_w)])

    def per_centroid(s, carry):
        base16 = (s // 16) * 16
        lanem = iota == (s - base16)
        neg = jnp.float32(-3e38)

        def bcast(off):
            chunk = cent_v[pl.ds(off + base16, 16)]
            return jnp.full((16,), jnp.max(jnp.where(lanem, chunk, neg)))

        cxv = bcast(0)
        cyv = bcast(s_per_w)
        czv = bcast(2 * s_per_w)

        def scan_cond(jc):
            j, cnt = jc
            return jnp.logical_and(cnt < NSAMPLE, j < N)

        def scan_body(jc):
            j, cnt = jc
            for u in range(4):
                jj = j + 16 * u
                px = xs_v[pl.ds(jj, 16)]
                py = ys_v[pl.ds(jj, 16)]
                pz = zs_v[pl.ds(jj, 16)]
                dx = px - cxv
                dy = py - cyv
                dz = pz - czv
                d2 = dx * dx + dy * dy
                d2 = d2 + dz * dz
                mask = d2 < r2
                incl = jnp.cumsum(jnp.where(mask, 1, 0).astype(jnp.int32))
                posn = cnt + incl - 1
                plsc.store_scatter(gbuf, [posn], jj + iota, mask=mask)
                cnt = cnt + jnp.max(incl)
            return j + 64, cnt

        _, cnt = lax.while_loop(scan_cond, scan_body, (jnp.int32(0), jnp.int32(0)))

        v0 = gbuf[pl.ds(0, 16)]
        v1 = gbuf[pl.ds(16, 16)]
        first = jnp.max(jnp.where(iota == 0, v0, -1))
        fb = jnp.full((16,), first, jnp.int32)
        boff = b * N
        out0 = jnp.where(iota < cnt, v0, fb) + boff
        out1 = jnp.where(iota + 16 < cnt, v1, fb) + boff
        idxbuf[pl.ds(0, 16)] = out0
        idxbuf[pl.ds(16, 16)] = out1

        pltpu.async_copy(table_hbm.at[idxbuf], rows_v, sem).wait()
        rowbase = (cbase + s) * NSAMPLE
        pltpu.sync_copy(rows_v, out_hbm.at[pl.ds(rowbase, NSAMPLE)])
        return carry

    lax.fori_loop(0, s_per_w, per_centroid, jnp.int32(0))


def _sc_call(coords, cents, table):
    mesh = plsc.VectorSubcoreMesh(core_axis_name="c", subcore_axis_name="s")
    fn = pl.kernel(
        _sc_body,
        mesh=mesh,
        compiler_params=pltpu.CompilerParams(needs_layout_passes=False),
        out_type=jax.ShapeDtypeStruct((B * NPOINT * NSAMPLE, 128), jnp.float32),
        scratch_types=[
            pltpu.VMEM((N,), jnp.float32),
            pltpu.VMEM((N,), jnp.float32),
            pltpu.VMEM((N,), jnp.float32),
            pltpu.VMEM((3 * 128,), jnp.float32),
            pltpu.VMEM((96,), jnp.int32),
            pltpu.VMEM((NSAMPLE,), jnp.int32),
            pltpu.VMEM((NSAMPLE, 128), jnp.float32),
            pltpu.SemaphoreType.DMA,
        ],
    )
    xs, ys, zs = (coords[:, i, :].reshape(B * N) for i in range(3))
    return fn(xs, ys, zs, cents[0], cents[1], cents[2], table)


def _mlp_body(sblk, g_ref, c_ref, w1c_ref, b1_ref, w2_ref, b2_ref,
              w3_ref, b3_ref, o_ref):
    q = lax.dot_general(c_ref[...], w1c_ref[...], (((0,), (0,)), ((), ())),
                        preferred_element_type=jnp.float32)
    qb = q - b1_ref[...]
    g3 = g_ref[...].reshape(sblk, NSAMPLE, 128)
    x1 = jnp.maximum(g3 - qb[:, None, :], 0.0).reshape(sblk * NSAMPLE, 128)
    x2 = jnp.maximum(
        jnp.dot(x1, w2_ref[...], preferred_element_type=jnp.float32)
        + b2_ref[...], 0.0)
    x3 = jnp.maximum(
        jnp.dot(x2, w3_ref[...], preferred_element_type=jnp.float32)
        + b3_ref[...], 0.0)
    o_ref[...] = jnp.max(x3.reshape(sblk, NSAMPLE, 128), axis=1)


def _mlp_call(G, cents, W1cs, b1r, W2s, b2r, W3s, b3r):
    sblk = 256
    ns = NSAMPLE
    stot = B * NPOINT
    grid = (stot // sblk,)
    return pl.pallas_call(
        functools.partial(_mlp_body, sblk),
        grid=grid,
        in_specs=[
            pl.BlockSpec((sblk * ns, 128), lambda i: (i, 0)),
            pl.BlockSpec((3, sblk), lambda i: (0, i)),
            pl.BlockSpec((3, 128), lambda i: (0, 0)),
            pl.BlockSpec((1, 128), lambda i: (0, 0)),
            pl.BlockSpec((128, 64), lambda i: (0, 0)),
            pl.BlockSpec((1, 64), lambda i: (0, 0)),
            pl.BlockSpec((64, 128), lambda i: (0, 0)),
            pl.BlockSpec((1, 128), lambda i: (0, 0)),
        ],
        out_specs=pl.BlockSpec((sblk, 128), lambda i: (i, 0)),
        out_shape=jax.ShapeDtypeStruct((stot, 128), jnp.float32),
    )(G, cents, W1cs, b1r, W2s, b2r, W3s, b3r)


def kernel(xyz, features, W1, g1, b1, W2, g2, b2, W3, g3, b3):
    k = (1.0 / jnp.sqrt(jnp.float32(1.0 + EPS))).astype(jnp.float32)
    coords = xyz.transpose(0, 2, 1)
    xyz_p = coords.reshape(B, 3, NROW, NCOL)

    nx, ny, nz = _fps_call(xyz_p)
    new_xyz = jnp.stack([nx, ny, nz], axis=-1)
    cents = jnp.concatenate(
        [nx.reshape(1, B * NPOINT), ny.reshape(1, B * NPOINT),
         nz.reshape(1, B * NPOINT)], axis=0)

    feats_t = features.transpose(0, 2, 1)
    pts = jnp.concatenate([xyz, feats_t], axis=-1)
    scale1 = (k * g1)[:, None]
    W1s = jnp.pad((W1 * scale1).T, ((0, 0), (0, 64)))
    A = _pts_call(pts, W1s).reshape(B * N, 128)

    G = _sc_call(coords, cents, A)

    W1cs = jnp.pad((W1[:, :3] * scale1).T, ((0, 0), (0, 64)))
    W2s = jnp.pad((W2 * (k * g2)[:, None]).T, ((0, 64), (0, 0)))
    W3s = (W3 * (k * g3)[:, None]).T
    out = _mlp_call(G, cents, W1cs, jnp.pad(b1, (0, 64)).reshape(1, 128), W2s,
                    b2.reshape(1, 64), W3s, b3.reshape(1, 128))
    new_features = out.reshape(B, NPOINT, 128).transpose(0, 2, 1)
    return new_xyz, new_features


# speedup vs baseline: 28.3674x; 1.1236x over previous
import functools

import jax
import jax.numpy as jnp
from jax import lax
from jax.experimental import pallas as pl
from jax.experimental.pallas import tpu as pltpu
from jax.experimental.pallas import tpu_sc as plsc

B, N, C = 4, 8192, 64
NPOINT, RADIUS, NSAMPLE = 1024, 0.4, 32
EPS = 1e-5
NROW, NCOL = 64, 128


def _fps_body(nb, nrow, ncol, npoint, xyz_ref, nx_ref, ny_ref, nz_ref,
              dists_ref):
    big = jnp.int32(1 << 30)
    rows1 = lax.broadcasted_iota(jnp.int32, (nrow, ncol), 0)
    col8 = lax.broadcasted_iota(jnp.int32, (8, ncol), 1)
    lane = lax.broadcasted_iota(jnp.int32, (1, ncol), 1)

    def comb(ta, tb):
        da, ra, xa, ya, za = ta
        db, rb, xb, yb, zb = tb
        take = (da > db) | ((da == db) & (ra < rb))
        f = lambda u, v: jnp.where(take, u, v)
        return f(da, db), f(ra, rb), f(xa, xb), f(ya, yb), f(za, zb)

    def tree_argmax(D, X, Y, Z):
        t = (D, rows1, X, Y, Z)
        r = nrow
        while r > 8:
            h = r // 2
            t = comb(tuple(a[:h] for a in t), tuple(a[h:] for a in t))
            r = h
        Ds, Rs, Xs, Ys, Zs = t
        def red2(op, a):
            return op(op(a, axis=0, keepdims=True), axis=1, keepdims=True)

        fl = Rs * ncol + col8
        m = red2(jnp.max, Ds)
        cand = jnp.where(Ds == m, fl, big)
        wi = red2(jnp.min, cand)
        sel = fl == wi
        wx = red2(jnp.sum, jnp.where(sel, Xs, 0.0))
        wy = red2(jnp.sum, jnp.where(sel, Ys, 0.0))
        wz = red2(jnp.sum, jnp.where(sel, Zs, 0.0))
        return wx, wy, wz

    def body(i, st):
        ws, chunks = st
        sel = lane == (i & (ncol - 1))
        k = i >> 7
        new_ws = []
        new_chunks = []
        for b in range(nb):
            wx, wy, wz = ws[b]
            cx, cy, cz = chunks[b]
            cx = jnp.where(sel, wx, cx)
            cy = jnp.where(sel, wy, cy)
            cz = jnp.where(sel, wz, cz)
            nx_ref[b, pl.ds(k, 1)] = cx.reshape(1, 1, ncol)
            ny_ref[b, pl.ds(k, 1)] = cy.reshape(1, 1, ncol)
            nz_ref[b, pl.ds(k, 1)] = cz.reshape(1, 1, ncol)
            new_chunks.append((cx, cy, cz))
            X = xyz_ref[b, 0]
            Y = xyz_ref[b, 1]
            Z = xyz_ref[b, 2]
            dx = X - wx
            dy = Y - wy
            dz = Z - wz
            d = dx * dx + dy * dy
            d = d + dz * dz
            dists = jnp.minimum(dists_ref[b], d)
            dists_ref[b] = dists
            new_ws.append(tree_argmax(dists, X, Y, Z))
        return tuple(new_ws), tuple(new_chunks)

    ws0 = []
    for b in range(nb):
        dists_ref[b] = jnp.full((nrow, ncol), 1e10, dtype=jnp.float32)
        p0 = (rows1 * ncol + lax.broadcasted_iota(jnp.int32, (nrow, ncol), 1)) == 0

        def red0(a):
            return jnp.sum(jnp.sum(a, axis=0, keepdims=True), axis=1,
                           keepdims=True)

        wx0 = red0(jnp.where(p0, xyz_ref[b, 0], 0.0))
        wy0 = red0(jnp.where(p0, xyz_ref[b, 1], 0.0))
        wz0 = red0(jnp.where(p0, xyz_ref[b, 2], 0.0))
        ws0.append((wx0, wy0, wz0))
    zc = jnp.zeros((1, ncol), dtype=jnp.float32)
    st = (tuple(ws0), tuple((zc, zc, zc) for _ in range(nb)))
    lax.fori_loop(0, npoint, body, st)


def _fps_call(xyz_p):
    nb = xyz_p.shape[0]
    out = jax.ShapeDtypeStruct((nb, NPOINT // NCOL, 1, NCOL), jnp.float32)
    nx, ny, nz = pl.pallas_call(
        functools.partial(_fps_body, nb, NROW, NCOL, NPOINT),
        out_shape=[out, out, out],
        scratch_shapes=[pltpu.VMEM((nb, NROW, NCOL), jnp.float32)],
    )(xyz_p)
    return (nx.reshape(nb, NPOINT), ny.reshape(nb, NPOINT),
            nz.reshape(nb, NPOINT))


def _pts_body(p_ref, w_ref, a_ref):
    a_ref[0] = jnp.dot(p_ref[0], w_ref[...], preferred_element_type=jnp.float32)


def _pts_call(pts, W1s):
    nb, n, ci = pts.shape
    co = W1s.shape[1]
    return pl.pallas_call(
        _pts_body,
        grid=(nb,),
        in_specs=[
            pl.BlockSpec((1, n, ci), lambda i: (i, 0, 0)),
            pl.BlockSpec((ci, co), lambda i: (0, 0)),
        ],
        out_specs=pl.BlockSpec((1, n, co), lambda i: (i, 0, 0)),
        out_shape=jax.ShapeDtypeStruct((nb, n, co), jnp.float32),
    )(pts, W1s)


def _sc_body(xs_hbm, ys_hbm, zs_hbm, cx_hbm, cy_hbm, cz_hbm, table_hbm, out_hbm,
             xs_v, ys_v, zs_v, cent_v, gbuf, idxbuf0, idxbuf1, rows0, rows1,
             gsem0, gsem1, wsem0, wsem1):
    nw = 32
    s_per_w = (B * NPOINT) // nw
    wid = lax.axis_index("s") * 2 + lax.axis_index("c")
    b = wid // (nw // B)
    cbase = wid * s_per_w
    r2 = jnp.float32(RADIUS * RADIUS)
    iota = lax.iota(jnp.int32, 16)

    pltpu.sync_copy(xs_hbm.at[pl.ds(b * N, N)], xs_v)
    pltpu.sync_copy(ys_hbm.at[pl.ds(b * N, N)], ys_v)
    pltpu.sync_copy(zs_hbm.at[pl.ds(b * N, N)], zs_v)
    for c, ref in enumerate((cx_hbm, cy_hbm, cz_hbm)):
        pltpu.sync_copy(ref.at[pl.ds(cbase, s_per_w)],
                        cent_v.at[pl.ds(c * s_per_w, s_per_w)])

    def scan(s, ibuf):
        base16 = (s // 16) * 16
        lanem = iota == (s - base16)
        neg = jnp.float32(-3e38)

        def bcast(off):
            chunk = cent_v[pl.ds(off + base16, 16)]
            return jnp.full((16,), jnp.max(jnp.where(lanem, chunk, neg)))

        cxv = bcast(0)
        cyv = bcast(s_per_w)
        czv = bcast(2 * s_per_w)

        def scan_cond(jc):
            j, cnt_vec = jc
            return jnp.logical_and(jnp.max(cnt_vec) < NSAMPLE, j < N)

        def scan_body(jc):
            j, cnt_vec = jc
            base_vec = cnt_vec
            for u in range(16):
                jj = j + 16 * u
                px = xs_v[pl.ds(jj, 16)]
                py = ys_v[pl.ds(jj, 16)]
                pz = zs_v[pl.ds(jj, 16)]
                dx = px - cxv
                dy = py - cyv
                dz = pz - czv
                d2 = dx * dx + dy * dy
                d2 = d2 + dz * dz
                mask = d2 < r2
                mi = jnp.where(mask, jnp.int32(1), jnp.int32(0))
                incl = plsc.cumsum(mi)
                posn = base_vec + incl - 1
                plsc.store_scatter(gbuf, [posn], jj + iota, mask=mask)
                base_vec = base_vec + plsc.all_reduce_population_count(mask)
            return j + 256, base_vec

        z16 = jnp.zeros((16,), jnp.int32)
        _, cnt_vec = lax.while_loop(scan_cond, scan_body, (jnp.int32(0), z16))
        cnt = jnp.max(cnt_vec)

        v0 = gbuf[pl.ds(0, 16)]
        v1 = gbuf[pl.ds(16, 16)]
        first = jnp.max(jnp.where(iota == 0, v0, -1))
        fb = jnp.full((16,), first, jnp.int32)
        boff = b * N
        ibuf[pl.ds(0, 16)] = jnp.where(iota < cnt, v0, fb) + boff
        ibuf[pl.ds(16, 16)] = jnp.where(iota + 16 < cnt, v1, fb) + boff

    def out_slice(s):
        return out_hbm.at[pl.ds((cbase + s) * NSAMPLE, NSAMPLE)]

    def pair_body(p, carry):
        s0 = 2 * p
        s1 = s0 + 1
        scan(s0, idxbuf0)

        @pl.when(p > 0)
        def _():
            pltpu.make_async_copy(rows0, out_slice(0), wsem0).wait()

        pltpu.async_copy(table_hbm.at[idxbuf0], rows0, gsem0)
        scan(s1, idxbuf1)

        @pl.when(p > 0)
        def _():
            pltpu.make_async_copy(rows1, out_slice(0), wsem1).wait()

        pltpu.async_copy(table_hbm.at[idxbuf1], rows1, gsem1)
        pltpu.make_async_copy(table_hbm.at[idxbuf0], rows0, gsem0).wait()
        pltpu.async_copy(rows0, out_slice(s0), wsem0)
        pltpu.make_async_copy(table_hbm.at[idxbuf1], rows1, gsem1).wait()
        pltpu.async_copy(rows1, out_slice(s1), wsem1)
        return carry

    lax.fori_loop(0, s_per_w // 2, pair_body, jnp.int32(0))
    pltpu.make_async_copy(rows0, out_slice(0), wsem0).wait()
    pltpu.make_async_copy(rows1, out_slice(0), wsem1).wait()


def _sc_call(coords, cents, table):
    mesh = plsc.VectorSubcoreMesh(core_axis_name="c", subcore_axis_name="s")
    fn = pl.kernel(
        _sc_body,
        mesh=mesh,
        compiler_params=pltpu.CompilerParams(needs_layout_passes=False),
        out_type=jax.ShapeDtypeStruct((B * NPOINT * NSAMPLE, 128), jnp.float32),
        scratch_types=[
            pltpu.VMEM((N,), jnp.float32),
            pltpu.VMEM((N,), jnp.float32),
            pltpu.VMEM((N,), jnp.float32),
            pltpu.VMEM((3 * 128,), jnp.float32),
            pltpu.VMEM((288,), jnp.int32),
            pltpu.VMEM((NSAMPLE,), jnp.int32),
            pltpu.VMEM((NSAMPLE,), jnp.int32),
            pltpu.VMEM((NSAMPLE, 128), jnp.float32),
            pltpu.VMEM((NSAMPLE, 128), jnp.float32),
            pltpu.SemaphoreType.DMA,
            pltpu.SemaphoreType.DMA,
            pltpu.SemaphoreType.DMA,
            pltpu.SemaphoreType.DMA,
        ],
    )
    xs, ys, zs = (coords[:, i, :].reshape(B * N) for i in range(3))
    return fn(xs, ys, zs, cents[0], cents[1], cents[2], table)


def _mlp_body(sblk, g_ref, c_ref, w1c_ref, b1_ref, w2_ref, b2_ref,
              w3_ref, b3_ref, o_ref):
    q = lax.dot_general(c_ref[...], w1c_ref[...], (((0,), (0,)), ((), ())),
                        preferred_element_type=jnp.float32)
    qb = q - b1_ref[...]
    g3 = g_ref[...].reshape(sblk, NSAMPLE, 128)
    x1 = jnp.maximum(g3 - qb[:, None, :], 0.0).reshape(sblk * NSAMPLE, 128)
    x2 = jnp.maximum(
        jnp.dot(x1, w2_ref[...], preferred_element_type=jnp.float32)
        + b2_ref[...], 0.0)
    x3 = jnp.maximum(
        jnp.dot(x2, w3_ref[...], preferred_element_type=jnp.float32)
        + b3_ref[...], 0.0)
    o_ref[...] = jnp.max(x3.reshape(sblk, NSAMPLE, 128), axis=1)


def _mlp_call(G, cents, W1cs, b1r, W2s, b2r, W3s, b3r):
    sblk = 256
    ns = NSAMPLE
    stot = B * NPOINT
    grid = (stot // sblk,)
    return pl.pallas_call(
        functools.partial(_mlp_body, sblk),
        grid=grid,
        in_specs=[
            pl.BlockSpec((sblk * ns, 128), lambda i: (i, 0)),
            pl.BlockSpec((3, sblk), lambda i: (0, i)),
            pl.BlockSpec((3, 128), lambda i: (0, 0)),
            pl.BlockSpec((1, 128), lambda i: (0, 0)),
            pl.BlockSpec((128, 64), lambda i: (0, 0)),
            pl.BlockSpec((1, 64), lambda i: (0, 0)),
            pl.BlockSpec((64, 128), lambda i: (0, 0)),
            pl.BlockSpec((1, 128), lambda i: (0, 0)),
        ],
        out_specs=pl.BlockSpec((sblk, 128), lambda i: (i, 0)),
        out_shape=jax.ShapeDtypeStruct((stot, 128), jnp.float32),
    )(G, cents, W1cs, b1r, W2s, b2r, W3s, b3r)


def kernel(xyz, features, W1, g1, b1, W2, g2, b2, W3, g3, b3):
    k = (1.0 / jnp.sqrt(jnp.float32(1.0 + EPS))).astype(jnp.float32)
    coords = xyz.transpose(0, 2, 1)
    xyz_p = coords.reshape(B, 3, NROW, NCOL)

    nx, ny, nz = _fps_call(xyz_p)
    new_xyz = jnp.stack([nx, ny, nz], axis=-1)
    cents = jnp.concatenate(
        [nx.reshape(1, B * NPOINT), ny.reshape(1, B * NPOINT),
         nz.reshape(1, B * NPOINT)], axis=0)

    feats_t = features.transpose(0, 2, 1)
    pts = jnp.concatenate([xyz, feats_t], axis=-1)
    scale1 = (k * g1)[:, None]
    W1s = jnp.pad((W1 * scale1).T, ((0, 0), (0, 64)))
    A = _pts_call(pts, W1s).reshape(B * N, 128)

    G = _sc_call(coords, cents, A)

    W1cs = jnp.pad((W1[:, :3] * scale1).T, ((0, 0), (0, 64)))
    W2s = jnp.pad((W2 * (k * g2)[:, None]).T, ((0, 64), (0, 0)))
    W3s = (W3 * (k * g3)[:, None]).T
    out = _mlp_call(G, cents, W1cs, jnp.pad(b1, (0, 64)).reshape(1, 128), W2s,
                    b2.reshape(1, 64), W3s, b3.reshape(1, 128))
    new_features = out.reshape(B, NPOINT, 128).transpose(0, 2, 1)
    return new_xyz, new_features


# speedup vs baseline: 29.0688x; 1.0247x over previous
import functools

import jax
import jax.numpy as jnp
from jax import lax
from jax.experimental import pallas as pl
from jax.experimental.pallas import tpu as pltpu
from jax.experimental.pallas import tpu_sc as plsc

B, N, C = 4, 8192, 64
NPOINT, RADIUS, NSAMPLE = 1024, 0.4, 32
EPS = 1e-5
NROW, NCOL = 64, 128


def _fps_body(nb, nrow, ncol, npoint, xyz_ref, nx_ref, ny_ref, nz_ref,
              dists_ref):
    big = jnp.int32(1 << 30)
    rows1 = lax.broadcasted_iota(jnp.int32, (nrow, ncol), 0)
    col8 = lax.broadcasted_iota(jnp.int32, (8, ncol), 1)
    lane = lax.broadcasted_iota(jnp.int32, (1, ncol), 1)

    def comb(ta, tb):
        da, ra, xa, ya, za = ta
        db, rb, xb, yb, zb = tb
        take = (da > db) | ((da == db) & (ra < rb))
        f = lambda u, v: jnp.where(take, u, v)
        return f(da, db), f(ra, rb), f(xa, xb), f(ya, yb), f(za, zb)

    def tree_prefix(D, X, Y, Z):
        t = (D, rows1, X, Y, Z)
        r = nrow
        while r > 8:
            h = r // 2
            t = comb(tuple(a[:h] for a in t), tuple(a[h:] for a in t))
            r = h
        return t

    def small_argmax(ts):
        def red2(op, a):
            return op(op(a, axis=2, keepdims=True), axis=1, keepdims=True)

        Ds, Rs, Xs, Ys, Zs = (jnp.stack([t[j] for t in ts]) for j in range(5))
        fl = Rs * ncol + col8[None]
        m = red2(jnp.max, Ds)
        cand = jnp.where(Ds == m, fl, big)
        wi = red2(jnp.min, cand)
        sel = fl == wi
        wx = red2(jnp.sum, jnp.where(sel, Xs, 0.0))
        wy = red2(jnp.sum, jnp.where(sel, Ys, 0.0))
        wz = red2(jnp.sum, jnp.where(sel, Zs, 0.0))
        return [(wx[b], wy[b], wz[b]) for b in range(len(ts))]

    def body(i, st):
        ws, chunks = st
        sel = lane == (i & (ncol - 1))
        k = i >> 7
        new_ws = []
        new_chunks = []
        dd = []
        for b in range(nb):
            wx, wy, wz = ws[b]
            cx, cy, cz = chunks[b]
            cx = jnp.where(sel, wx, cx)
            cy = jnp.where(sel, wy, cy)
            cz = jnp.where(sel, wz, cz)
            nx_ref[b, pl.ds(k, 1)] = cx.reshape(1, 1, ncol)
            ny_ref[b, pl.ds(k, 1)] = cy.reshape(1, 1, ncol)
            nz_ref[b, pl.ds(k, 1)] = cz.reshape(1, 1, ncol)
            new_chunks.append((cx, cy, cz))
            X = xyz_ref[b, 0]
            Y = xyz_ref[b, 1]
            Z = xyz_ref[b, 2]
            dx = X - wx
            dy = Y - wy
            dz = Z - wz
            d = dx * dx + dy * dy
            d = d + dz * dz
            dists = jnp.minimum(dists_ref[b], d)
            dists_ref[b] = dists
            dd.append((dists, X, Y, Z))
        ts = [tree_prefix(*dd[b]) for b in range(nb)]
        new_ws = small_argmax(ts)
        return tuple(new_ws), tuple(new_chunks)

    ws0 = []
    for b in range(nb):
        dists_ref[b] = jnp.full((nrow, ncol), 1e10, dtype=jnp.float32)
        p0 = (rows1 * ncol + lax.broadcasted_iota(jnp.int32, (nrow, ncol), 1)) == 0

        def red0(a):
            return jnp.sum(jnp.sum(a, axis=0, keepdims=True), axis=1,
                           keepdims=True)

        wx0 = red0(jnp.where(p0, xyz_ref[b, 0], 0.0))
        wy0 = red0(jnp.where(p0, xyz_ref[b, 1], 0.0))
        wz0 = red0(jnp.where(p0, xyz_ref[b, 2], 0.0))
        ws0.append((wx0, wy0, wz0))
    zc = jnp.zeros((1, ncol), dtype=jnp.float32)
    st = (tuple(ws0), tuple((zc, zc, zc) for _ in range(nb)))
    lax.fori_loop(0, npoint, body, st)


def _fps_call(xyz_p):
    nb = xyz_p.shape[0]
    out = jax.ShapeDtypeStruct((nb, NPOINT // NCOL, 1, NCOL), jnp.float32)
    nx, ny, nz = pl.pallas_call(
        functools.partial(_fps_body, nb, NROW, NCOL, NPOINT),
        out_shape=[out, out, out],
        scratch_shapes=[pltpu.VMEM((nb, NROW, NCOL), jnp.float32)],
    )(xyz_p)
    return (nx.reshape(nb, NPOINT), ny.reshape(nb, NPOINT),
            nz.reshape(nb, NPOINT))


def _pts_body(p_ref, w_ref, a_ref):
    a_ref[0] = jnp.dot(p_ref[0], w_ref[...], preferred_element_type=jnp.float32)


def _pts_call(pts, W1s):
    nb, n, ci = pts.shape
    co = W1s.shape[1]
    return pl.pallas_call(
        _pts_body,
        grid=(nb,),
        in_specs=[
            pl.BlockSpec((1, n, ci), lambda i: (i, 0, 0)),
            pl.BlockSpec((ci, co), lambda i: (0, 0)),
        ],
        out_specs=pl.BlockSpec((1, n, co), lambda i: (i, 0, 0)),
        out_shape=jax.ShapeDtypeStruct((nb, n, co), jnp.float32),
    )(pts, W1s)


def _sc_body(xs_hbm, ys_hbm, zs_hbm, cx_hbm, cy_hbm, cz_hbm, table_hbm, out_hbm,
             xs_v, ys_v, zs_v, cent_v, gbuf, idxbuf0, idxbuf1, rows0, rows1,
             gsem0, gsem1, wsem0, wsem1):
    nw = 32
    s_per_w = (B * NPOINT) // nw
    wid = lax.axis_index("s") * 2 + lax.axis_index("c")
    b = wid // (nw // B)
    cbase = wid * s_per_w
    r2 = jnp.float32(RADIUS * RADIUS)
    iota = lax.iota(jnp.int32, 16)

    pltpu.sync_copy(xs_hbm.at[pl.ds(b * N, N)], xs_v)
    pltpu.sync_copy(ys_hbm.at[pl.ds(b * N, N)], ys_v)
    pltpu.sync_copy(zs_hbm.at[pl.ds(b * N, N)], zs_v)
    for c, ref in enumerate((cx_hbm, cy_hbm, cz_hbm)):
        pltpu.sync_copy(ref.at[pl.ds(cbase, s_per_w)],
                        cent_v.at[pl.ds(c * s_per_w, s_per_w)])

    def scan(s, ibuf, off):
        base16 = (s // 16) * 16
        lanem = iota == (s - base16)
        neg = jnp.float32(-3e38)

        def bcast(off):
            chunk = cent_v[pl.ds(off + base16, 16)]
            return jnp.full((16,), jnp.max(jnp.where(lanem, chunk, neg)))

        cxv = bcast(0)
        cyv = bcast(s_per_w)
        czv = bcast(2 * s_per_w)

        def scan_cond(jc):
            j, cnt_vec = jc
            return jnp.logical_and(jnp.max(cnt_vec) < NSAMPLE, j < N)

        def scan_body(jc):
            j, cnt_vec = jc
            base_vec = cnt_vec
            for u in range(16):
                jj = j + 16 * u
                px = xs_v[pl.ds(jj, 16)]
                py = ys_v[pl.ds(jj, 16)]
                pz = zs_v[pl.ds(jj, 16)]
                dx = px - cxv
                dy = py - cyv
                dz = pz - czv
                d2 = dx * dx + dy * dy
                d2 = d2 + dz * dz
                mask = d2 < r2
                mi = jnp.where(mask, jnp.int32(1), jnp.int32(0))
                incl = plsc.cumsum(mi)
                posn = base_vec + incl - 1
                plsc.store_scatter(gbuf, [posn], jj + iota, mask=mask)
                base_vec = base_vec + plsc.all_reduce_population_count(mask)
            return j + 256, base_vec

        z16 = jnp.zeros((16,), jnp.int32)
        _, cnt_vec = lax.while_loop(scan_cond, scan_body, (jnp.int32(0), z16))
        cnt = jnp.max(cnt_vec)

        v0 = gbuf[pl.ds(0, 16)]
        v1 = gbuf[pl.ds(16, 16)]
        first = jnp.max(jnp.where(iota == 0, v0, -1))
        fb = jnp.full((16,), first, jnp.int32)
        boff = b * N
        ibuf[pl.ds(off, 16)] = jnp.where(iota < cnt, v0, fb) + boff
        ibuf[pl.ds(off + 16, 16)] = jnp.where(iota + 16 < cnt, v1, fb) + boff

    def out_slice(s):
        return out_hbm.at[pl.ds((cbase + s) * NSAMPLE, 8 * NSAMPLE)]

    grp = 8

    def pair_body(p, carry):
        g0 = 2 * p
        g1 = g0 + 1
        for c in range(grp):
            scan(g0 * grp + c, idxbuf0, c * NSAMPLE)

        @pl.when(p > 0)
        def _():
            pltpu.make_async_copy(rows0, out_slice(0), wsem0).wait()

        pltpu.async_copy(table_hbm.at[idxbuf0], rows0, gsem0)
        for c in range(grp):
            scan(g1 * grp + c, idxbuf1, c * NSAMPLE)

        @pl.when(p > 0)
        def _():
            pltpu.make_async_copy(rows1, out_slice(0), wsem1).wait()

        pltpu.async_copy(table_hbm.at[idxbuf1], rows1, gsem1)
        pltpu.make_async_copy(table_hbm.at[idxbuf0], rows0, gsem0).wait()
        pltpu.async_copy(rows0, out_slice(g0 * grp), wsem0)
        pltpu.make_async_copy(table_hbm.at[idxbuf1], rows1, gsem1).wait()
        pltpu.async_copy(rows1, out_slice(g1 * grp), wsem1)
        return carry

    lax.fori_loop(0, s_per_w // (2 * grp), pair_body, jnp.int32(0))
    pltpu.make_async_copy(rows0, out_slice(0), wsem0).wait()
    pltpu.make_async_copy(rows1, out_slice(0), wsem1).wait()


def _sc_call(coords, cents, table):
    mesh = plsc.VectorSubcoreMesh(core_axis_name="c", subcore_axis_name="s")
    fn = pl.kernel(
        _sc_body,
        mesh=mesh,
        compiler_params=pltpu.CompilerParams(needs_layout_passes=False),
        out_type=jax.ShapeDtypeStruct((B * NPOINT * NSAMPLE, 128), jnp.float32),
        scratch_types=[
            pltpu.VMEM((N,), jnp.float32),
            pltpu.VMEM((N,), jnp.float32),
            pltpu.VMEM((N,), jnp.float32),
            pltpu.VMEM((3 * 128,), jnp.float32),
            pltpu.VMEM((288,), jnp.int32),
            pltpu.VMEM((8 * NSAMPLE,), jnp.int32),
            pltpu.VMEM((8 * NSAMPLE,), jnp.int32),
            pltpu.VMEM((8 * NSAMPLE, 128), jnp.float32),
            pltpu.VMEM((8 * NSAMPLE, 128), jnp.float32),
            pltpu.SemaphoreType.DMA,
            pltpu.SemaphoreType.DMA,
            pltpu.SemaphoreType.DMA,
            pltpu.SemaphoreType.DMA,
        ],
    )
    xs, ys, zs = (coords[:, i, :].reshape(B * N) for i in range(3))
    return fn(xs, ys, zs, cents[0], cents[1], cents[2], table)


def _mlp_body(sblk, g_ref, c_ref, w1c_ref, b1_ref, w2_ref, b2_ref,
              w3_ref, b3_ref, o_ref):
    q = lax.dot_general(c_ref[...], w1c_ref[...], (((0,), (0,)), ((), ())),
                        preferred_element_type=jnp.float32)
    qb = q - b1_ref[...]
    g3 = g_ref[...].reshape(sblk, NSAMPLE, 128)
    x1 = jnp.maximum(g3 - qb[:, None, :], 0.0).reshape(sblk * NSAMPLE, 128)
    x2 = jnp.maximum(
        jnp.dot(x1, w2_ref[...], preferred_element_type=jnp.float32)
        + b2_ref[...], 0.0)
    x3 = jnp.maximum(
        jnp.dot(x2, w3_ref[...], preferred_element_type=jnp.float32)
        + b3_ref[...], 0.0)
    o_ref[...] = jnp.max(x3.reshape(sblk, NSAMPLE, 128), axis=1)


def _mlp_call(G, cents, W1cs, b1r, W2s, b2r, W3s, b3r):
    sblk = 256
    ns = NSAMPLE
    stot = B * NPOINT
    grid = (stot // sblk,)
    return pl.pallas_call(
        functools.partial(_mlp_body, sblk),
        grid=grid,
        in_specs=[
            pl.BlockSpec((sblk * ns, 128), lambda i: (i, 0)),
            pl.BlockSpec((3, sblk), lambda i: (0, i)),
            pl.BlockSpec((3, 128), lambda i: (0, 0)),
            pl.BlockSpec((1, 128), lambda i: (0, 0)),
            pl.BlockSpec((128, 64), lambda i: (0, 0)),
            pl.BlockSpec((1, 64), lambda i: (0, 0)),
            pl.BlockSpec((64, 128), lambda i: (0, 0)),
            pl.BlockSpec((1, 128), lambda i: (0, 0)),
        ],
        out_specs=pl.BlockSpec((sblk, 128), lambda i: (i, 0)),
        out_shape=jax.ShapeDtypeStruct((stot, 128), jnp.float32),
    )(G, cents, W1cs, b1r, W2s, b2r, W3s, b3r)


def kernel(xyz, features, W1, g1, b1, W2, g2, b2, W3, g3, b3):
    k = (1.0 / jnp.sqrt(jnp.float32(1.0 + EPS))).astype(jnp.float32)
    coords = xyz.transpose(0, 2, 1)
    xyz_p = coords.reshape(B, 3, NROW, NCOL)

    nx, ny, nz = _fps_call(xyz_p)
    new_xyz = jnp.stack([nx, ny, nz], axis=-1)
    cents = jnp.concatenate(
        [nx.reshape(1, B * NPOINT), ny.reshape(1, B * NPOINT),
         nz.reshape(1, B * NPOINT)], axis=0)

    feats_t = features.transpose(0, 2, 1)
    pts = jnp.concatenate([xyz, feats_t], axis=-1)
    scale1 = (k * g1)[:, None]
    W1s = jnp.pad((W1 * scale1).T, ((0, 0), (0, 64)))
    A = _pts_call(pts, W1s).reshape(B * N, 128)

    G = _sc_call(coords, cents, A)

    W1cs = jnp.pad((W1[:, :3] * scale1).T, ((0, 0), (0, 64)))
    W2s = jnp.pad((W2 * (k * g2)[:, None]).T, ((0, 64), (0, 0)))
    W3s = (W3 * (k * g3)[:, None]).T
    out = _mlp_call(G, cents, W1cs, jnp.pad(b1, (0, 64)).reshape(1, 128), W2s,
                    b2.reshape(1, 64), W3s, b3.reshape(1, 128))
    new_features = out.reshape(B, NPOINT, 128).transpose(0, 2, 1)
    return new_xyz, new_features


# speedup vs baseline: 29.0841x; 1.0005x over previous
import functools

import jax
import jax.numpy as jnp
from jax import lax
from jax.experimental import pallas as pl
from jax.experimental.pallas import tpu as pltpu
from jax.experimental.pallas import tpu_sc as plsc

B, N, C = 4, 8192, 64
NPOINT, RADIUS, NSAMPLE = 1024, 0.4, 32
EPS = 1e-5
NROW, NCOL = 64, 128


def _fps_body(nb, nrow, ncol, npoint, xyz_ref, nx_ref, ny_ref, nz_ref,
              dists_ref):
    big = jnp.int32(1 << 30)
    rows1 = lax.broadcasted_iota(jnp.int32, (nrow, ncol), 0)
    col8 = lax.broadcasted_iota(jnp.int32, (8, ncol), 1)
    lane = lax.broadcasted_iota(jnp.int32, (1, ncol), 1)

    def comb(ta, tb):
        da, ra, xa, ya, za = ta
        db, rb, xb, yb, zb = tb
        take = (da > db) | ((da == db) & (ra < rb))
        f = lambda u, v: jnp.where(take, u, v)
        return f(da, db), f(ra, rb), f(xa, xb), f(ya, yb), f(za, zb)

    def tree_prefix(D, X, Y, Z):
        t = (D, rows1, X, Y, Z)
        r = nrow
        while r > 8:
            h = r // 2
            t = comb(tuple(a[:h] for a in t), tuple(a[h:] for a in t))
            r = h
        return t

    def small_argmax(ts):
        def red2(op, a):
            return op(op(a, axis=2, keepdims=True), axis=1, keepdims=True)

        Ds, Rs, Xs, Ys, Zs = (jnp.stack([t[j] for t in ts]) for j in range(5))
        fl = Rs * ncol + col8[None]
        m = red2(jnp.max, Ds)
        cand = jnp.where(Ds == m, fl, big)
        wi = red2(jnp.min, cand)
        sel = fl == wi
        wx = red2(jnp.sum, jnp.where(sel, Xs, 0.0))
        wy = red2(jnp.sum, jnp.where(sel, Ys, 0.0))
        wz = red2(jnp.sum, jnp.where(sel, Zs, 0.0))
        return [(wx[b], wy[b], wz[b]) for b in range(len(ts))]

    def body(i, st):
        ws, chunks = st
        sel = lane == (i & (ncol - 1))
        k = i >> 7
        new_ws = []
        new_chunks = []
        dd = []
        for b in range(nb):
            wx, wy, wz = ws[b]
            cx, cy, cz = chunks[b]
            cx = jnp.where(sel, wx, cx)
            cy = jnp.where(sel, wy, cy)
            cz = jnp.where(sel, wz, cz)
            nx_ref[b, pl.ds(k, 1)] = cx.reshape(1, 1, ncol)
            ny_ref[b, pl.ds(k, 1)] = cy.reshape(1, 1, ncol)
            nz_ref[b, pl.ds(k, 1)] = cz.reshape(1, 1, ncol)
            new_chunks.append((cx, cy, cz))
            X = xyz_ref[b, 0]
            Y = xyz_ref[b, 1]
            Z = xyz_ref[b, 2]
            dx = X - wx
            dy = Y - wy
            dz = Z - wz
            d = dx * dx + dy * dy
            d = d + dz * dz
            dists = jnp.minimum(dists_ref[b], d)
            dists_ref[b] = dists
            dd.append((dists, X, Y, Z))
        ts = [tree_prefix(*dd[b]) for b in range(nb)]
        new_ws = small_argmax(ts)
        return tuple(new_ws), tuple(new_chunks)

    ws0 = []
    for b in range(nb):
        dists_ref[b] = jnp.full((nrow, ncol), 1e10, dtype=jnp.float32)
        p0 = (rows1 * ncol + lax.broadcasted_iota(jnp.int32, (nrow, ncol), 1)) == 0

        def red0(a):
            return jnp.sum(jnp.sum(a, axis=0, keepdims=True), axis=1,
                           keepdims=True)

        wx0 = red0(jnp.where(p0, xyz_ref[b, 0], 0.0))
        wy0 = red0(jnp.where(p0, xyz_ref[b, 1], 0.0))
        wz0 = red0(jnp.where(p0, xyz_ref[b, 2], 0.0))
        ws0.append((wx0, wy0, wz0))
    zc = jnp.zeros((1, ncol), dtype=jnp.float32)
    st = (tuple(ws0), tuple((zc, zc, zc) for _ in range(nb)))
    lax.fori_loop(0, npoint, body, st)


def _fps_call(xyz_p):
    nb = xyz_p.shape[0]
    out = jax.ShapeDtypeStruct((nb, NPOINT // NCOL, 1, NCOL), jnp.float32)
    nx, ny, nz = pl.pallas_call(
        functools.partial(_fps_body, nb, NROW, NCOL, NPOINT),
        out_shape=[out, out, out],
        scratch_shapes=[pltpu.VMEM((nb, NROW, NCOL), jnp.float32)],
    )(xyz_p)
    return (nx.reshape(nb, NPOINT), ny.reshape(nb, NPOINT),
            nz.reshape(nb, NPOINT))


def _pts_body(p_ref, w_ref, a_ref):
    a_ref[0] = jnp.dot(p_ref[0], w_ref[...], preferred_element_type=jnp.float32)


def _pts_call(pts, W1s):
    nb, n, ci = pts.shape
    co = W1s.shape[1]
    return pl.pallas_call(
        _pts_body,
        grid=(nb,),
        in_specs=[
            pl.BlockSpec((1, n, ci), lambda i: (i, 0, 0)),
            pl.BlockSpec((ci, co), lambda i: (0, 0)),
        ],
        out_specs=pl.BlockSpec((1, n, co), lambda i: (i, 0, 0)),
        out_shape=jax.ShapeDtypeStruct((nb, n, co), jnp.float32),
    )(pts, W1s)


def _sc_body(xs_hbm, ys_hbm, zs_hbm, cx_hbm, cy_hbm, cz_hbm, table_hbm, out_hbm,
             xs_v, ys_v, zs_v, cent_v, gbuf, idxbuf0a, idxbuf0b, idxbuf1a,
             idxbuf1b, rows0, rows1, gsem0, gsem1, wsem0, wsem1):
    nw = 32
    s_per_w = (B * NPOINT) // nw
    wid = lax.axis_index("s") * 2 + lax.axis_index("c")
    b = wid // (nw // B)
    cbase = wid * s_per_w
    r2 = jnp.float32(RADIUS * RADIUS)
    iota = lax.iota(jnp.int32, 16)

    pltpu.sync_copy(xs_hbm.at[pl.ds(b * N, N)], xs_v)
    pltpu.sync_copy(ys_hbm.at[pl.ds(b * N, N)], ys_v)
    pltpu.sync_copy(zs_hbm.at[pl.ds(b * N, N)], zs_v)
    for c, ref in enumerate((cx_hbm, cy_hbm, cz_hbm)):
        pltpu.sync_copy(ref.at[pl.ds(cbase, s_per_w)],
                        cent_v.at[pl.ds(c * s_per_w, s_per_w)])

    def scan(s, ibuf, off):
        base16 = (s // 16) * 16
        lanem = iota == (s - base16)
        neg = jnp.float32(-3e38)

        def bcast(off):
            chunk = cent_v[pl.ds(off + base16, 16)]
            return jnp.full((16,), jnp.max(jnp.where(lanem, chunk, neg)))

        cxv = bcast(0)
        cyv = bcast(s_per_w)
        czv = bcast(2 * s_per_w)

        def scan_cond(jc):
            j, cnt_vec = jc
            return jnp.logical_and(jnp.max(cnt_vec) < NSAMPLE, j < N)

        def scan_body(jc):
            j, cnt_vec = jc
            base_vec = cnt_vec
            for u in range(16):
                jj = j + 16 * u
                px = xs_v[pl.ds(jj, 16)]
                py = ys_v[pl.ds(jj, 16)]
                pz = zs_v[pl.ds(jj, 16)]
                dx = px - cxv
                dy = py - cyv
                dz = pz - czv
                d2 = dx * dx + dy * dy
                d2 = d2 + dz * dz
                mask = d2 < r2
                mi = jnp.where(mask, jnp.int32(1), jnp.int32(0))
                incl = plsc.cumsum(mi)
                posn = base_vec + incl - 1
                plsc.store_scatter(gbuf, [posn], jj + iota, mask=mask)
                base_vec = base_vec + plsc.all_reduce_population_count(mask)
            return j + 256, base_vec

        z16 = jnp.zeros((16,), jnp.int32)
        _, cnt_vec = lax.while_loop(scan_cond, scan_body, (jnp.int32(0), z16))
        cnt = jnp.max(cnt_vec)

        v0 = gbuf[pl.ds(0, 16)]
        v1 = gbuf[pl.ds(16, 16)]
        first = jnp.max(jnp.where(iota == 0, v0, -1))
        fb = jnp.full((16,), first, jnp.int32)
        boff = b * N
        ibuf[pl.ds(off, 16)] = jnp.where(iota < cnt, v0, fb) + boff
        ibuf[pl.ds(off + 16, 16)] = jnp.where(iota + 16 < cnt, v1, fb) + boff

    def out_slice(s):
        return out_hbm.at[pl.ds((cbase + s) * NSAMPLE, 8 * NSAMPLE)]

    grp = 8

    def scan_group(g, bufa, bufb):
        for c in range(grp):
            buf, off = (bufa, c * NSAMPLE) if c < 4 else (bufb,
                                                          (c - 4) * NSAMPLE)
            scan(g * grp + c, buf, off)

    def gather_group(bufa, bufb, rows, sem):
        pltpu.async_copy(table_hbm.at[bufa], rows.at[pl.ds(0, 128)], sem)
        pltpu.async_copy(table_hbm.at[bufb], rows.at[pl.ds(128, 128)], sem)

    def wait_gather(bufa, bufb, rows, sem):
        pltpu.make_async_copy(table_hbm.at[bufa], rows.at[pl.ds(0, 128)],
                              sem).wait()
        pltpu.make_async_copy(table_hbm.at[bufb], rows.at[pl.ds(128, 128)],
                              sem).wait()

    def pair_body(p, carry):
        g0 = 2 * p
        g1 = g0 + 1
        scan_group(g0, idxbuf0a, idxbuf0b)

        @pl.when(p > 0)
        def _():
            pltpu.make_async_copy(rows0, out_slice(0), wsem0).wait()

        gather_group(idxbuf0a, idxbuf0b, rows0, gsem0)
        scan_group(g1, idxbuf1a, idxbuf1b)

        @pl.when(p > 0)
        def _():
            pltpu.make_async_copy(rows1, out_slice(0), wsem1).wait()

        gather_group(idxbuf1a, idxbuf1b, rows1, gsem1)
        wait_gather(idxbuf0a, idxbuf0b, rows0, gsem0)
        pltpu.async_copy(rows0, out_slice(g0 * grp), wsem0)
        wait_gather(idxbuf1a, idxbuf1b, rows1, gsem1)
        pltpu.async_copy(rows1, out_slice(g1 * grp), wsem1)
        return carry

    lax.fori_loop(0, s_per_w // (2 * grp), pair_body, jnp.int32(0))
    pltpu.make_async_copy(rows0, out_slice(0), wsem0).wait()
    pltpu.make_async_copy(rows1, out_slice(0), wsem1).wait()


def _sc_call(coords, cents, table):
    mesh = plsc.VectorSubcoreMesh(core_axis_name="c", subcore_axis_name="s")
    fn = pl.kernel(
        _sc_body,
        mesh=mesh,
        compiler_params=pltpu.CompilerParams(needs_layout_passes=False),
        out_type=jax.ShapeDtypeStruct((B * NPOINT * NSAMPLE, 128), jnp.float32),
        scratch_types=[
            pltpu.VMEM((N,), jnp.float32),
            pltpu.VMEM((N,), jnp.float32),
            pltpu.VMEM((N,), jnp.float32),
            pltpu.VMEM((3 * 128,), jnp.float32),
            pltpu.VMEM((288,), jnp.int32),
            pltpu.VMEM((4 * NSAMPLE,), jnp.int32),
            pltpu.VMEM((4 * NSAMPLE,), jnp.int32),
            pltpu.VMEM((4 * NSAMPLE,), jnp.int32),
            pltpu.VMEM((4 * NSAMPLE,), jnp.int32),
            pltpu.VMEM((8 * NSAMPLE, 128), jnp.float32),
            pltpu.VMEM((8 * NSAMPLE, 128), jnp.float32),
            pltpu.SemaphoreType.DMA,
            pltpu.SemaphoreType.DMA,
            pltpu.SemaphoreType.DMA,
            pltpu.SemaphoreType.DMA,
        ],
    )
    xs, ys, zs = (coords[:, i, :].reshape(B * N) for i in range(3))
    return fn(xs, ys, zs, cents[0], cents[1], cents[2], table)


def _mlp_body(sblk, g_ref, c_ref, w1c_ref, b1_ref, w2_ref, b2_ref,
              w3_ref, b3_ref, o_ref):
    q = lax.dot_general(c_ref[...], w1c_ref[...], (((0,), (0,)), ((), ())),
                        preferred_element_type=jnp.float32)
    qb = q - b1_ref[...]
    g3 = g_ref[...].reshape(sblk, NSAMPLE, 128)
    x1 = jnp.maximum(g3 - qb[:, None, :], 0.0).reshape(sblk * NSAMPLE, 128)
    x2 = jnp.maximum(
        jnp.dot(x1, w2_ref[...], preferred_element_type=jnp.float32)
        + b2_ref[...], 0.0)
    x3 = jnp.maximum(
        jnp.dot(x2, w3_ref[...], preferred_element_type=jnp.float32)
        + b3_ref[...], 0.0)
    o_ref[...] = jnp.max(x3.reshape(sblk, NSAMPLE, 128), axis=1)


def _mlp_call(G, cents, W1cs, b1r, W2s, b2r, W3s, b3r):
    sblk = 256
    ns = NSAMPLE
    stot = B * NPOINT
    grid = (stot // sblk,)
    return pl.pallas_call(
        functools.partial(_mlp_body, sblk),
        grid=grid,
        in_specs=[
            pl.BlockSpec((sblk * ns, 128), lambda i: (i, 0)),
            pl.BlockSpec((3, sblk), lambda i: (0, i)),
            pl.BlockSpec((3, 128), lambda i: (0, 0)),
            pl.BlockSpec((1, 128), lambda i: (0, 0)),
            pl.BlockSpec((128, 64), lambda i: (0, 0)),
            pl.BlockSpec((1, 64), lambda i: (0, 0)),
            pl.BlockSpec((64, 128), lambda i: (0, 0)),
            pl.BlockSpec((1, 128), lambda i: (0, 0)),
        ],
        out_specs=pl.BlockSpec((sblk, 128), lambda i: (i, 0)),
        out_shape=jax.ShapeDtypeStruct((stot, 128), jnp.float32),
    )(G, cents, W1cs, b1r, W2s, b2r, W3s, b3r)


def kernel(xyz, features, W1, g1, b1, W2, g2, b2, W3, g3, b3):
    k = (1.0 / jnp.sqrt(jnp.float32(1.0 + EPS))).astype(jnp.float32)
    coords = xyz.transpose(0, 2, 1)
    xyz_p = coords.reshape(B, 3, NROW, NCOL)

    nx, ny, nz = _fps_call(xyz_p)
    new_xyz = jnp.stack([nx, ny, nz], axis=-1)
    cents = jnp.concatenate(
        [nx.reshape(1, B * NPOINT), ny.reshape(1, B * NPOINT),
         nz.reshape(1, B * NPOINT)], axis=0)

    feats_t = features.transpose(0, 2, 1)
    pts = jnp.concatenate([xyz, feats_t], axis=-1)
    scale1 = (k * g1)[:, None]
    W1s = jnp.pad((W1 * scale1).T, ((0, 0), (0, 64)))
    A = _pts_call(pts, W1s).reshape(B * N, 128)

    G = _sc_call(coords, cents, A)

    W1cs = jnp.pad((W1[:, :3] * scale1).T, ((0, 0), (0, 64)))
    W2s = jnp.pad((W2 * (k * g2)[:, None]).T, ((0, 64), (0, 0)))
    W3s = (W3 * (k * g3)[:, None]).T
    out = _mlp_call(G, cents, W1cs, jnp.pad(b1, (0, 64)).reshape(1, 128), W2s,
                    b2.reshape(1, 64), W3s, b3.reshape(1, 128))
    new_features = out.reshape(B, NPOINT, 128).transpose(0, 2, 1)
    return new_xyz, new_features
